# baseline (speedup 1.0000x reference)
import functools

import jax
import jax.numpy as jnp
from jax import lax
from jax.experimental import pallas as pl
from jax.experimental.pallas import tpu as pltpu
from jax.experimental.pallas import tpu_sc as plsc

TABLE_SIZE = 640
HIDDEN = 256
B = 4096
L = 50

_info = plsc.get_sparse_core_info()
NC, NS = _info.num_cores, _info.num_subcores
NW = NC * NS
SPT = B // NW
QS = 64
NQ = SPT // QS
PR = QS // 2

_mesh = plsc.VectorSubcoreMesh(core_axis_name="c", subcore_axis_name="s")


@functools.partial(
    pl.kernel,
    mesh=_mesh,
    out_type=jax.ShapeDtypeStruct((B // 2, TABLE_SIZE), jnp.int32),
    scratch_types=[
        pltpu.VMEM((SPT, L), jnp.int32),
        pltpu.VMEM((SPT, L), jnp.int32),
        pltpu.VMEM((PR, TABLE_SIZE), jnp.int32),
        pltpu.VMEM((PR, TABLE_SIZE), jnp.int32),
        pltpu.SemaphoreType.DMA,
        pltpu.SemaphoreType.DMA,
        pltpu.SemaphoreType.DMA,
    ],
    compiler_params=pltpu.CompilerParams(needs_layout_passes=False),
)
def _sc_hist(idx0_hbm, idx1_hbm, out_hbm, idx0_v, idx1_v, cnt_a, cnt_b,
             sem_i, sem_a, sem_b):
    wid = lax.axis_index("s") * NC + lax.axis_index("c")
    sbase = wid * SPT
    pbase = wid * (SPT // 2)
    zero16 = jnp.zeros((16,), jnp.int32)
    adds = [[jnp.full((16,), 1 << (16 * f + 8 * t), jnp.int32)
             for t in range(2)] for f in range(2)]
    lane = lax.iota(jnp.int32, 16)
    lane2 = lane * 2
    ld0 = pltpu.async_copy(idx0_hbm.at[pl.ds(sbase, SPT)], idx0_v, sem_i)
    ld1 = pltpu.async_copy(idx1_hbm.at[pl.ds(sbase, SPT)], idx1_v, sem_i)
    ld0.wait()
    ld1.wait()

    copies = []
    for q in range(NQ):
        cnt_v = (cnt_a, cnt_b)[q % 2]
        sem_o = (sem_a, sem_b)[q % 2]

        def zrow_body(s, zc, cnt_v=cnt_v):
            for k in range(TABLE_SIZE // 16):
                cnt_v[s, pl.ds(k * 16, 16)] = zero16
            return zc

        lax.fori_loop(0, PR, zrow_body, 0)

        rows = [lane + rh * 16 for rh in range(2)]
        svecs = [lane2 + (q * QS + 32 * rh + f)
                 for f in range(2) for rh in range(2)]

        def j_body(j, jc, cnt_v=cnt_v, rows=rows, svecs=svecs):
            jv = jnp.full((16,), 1, jnp.int32) * j
            for g in range(4):
                f, rh = g // 2, g % 2
                sv = svecs[g]
                cv0 = plsc.load_gather(idx0_v, [sv, jv])
                plsc.addupdate_scatter(cnt_v, [rows[rh], cv0], adds[f][0])
                cv1 = plsc.load_gather(idx1_v, [sv, jv])
                plsc.addupdate_scatter(cnt_v, [rows[rh], cv1], adds[f][1])
            return jc

        lax.fori_loop(0, L, j_body, 0)

        copies.append(pltpu.async_copy(
            cnt_v, out_hbm.at[pl.ds(pbase + q * PR, PR)], sem_o))
    for cp in copies:
        cp.wait()


BT2 = 512


def _tc_kernel(cnt_ref, emb0_ref, emb1_ref, w2_ref, b2_ref,
               w3_ref, b3_ref, w4_ref, b4_ref, out_ref):
    w = cnt_ref[...]
    c0 = jnp.concatenate(
        [jnp.bitwise_and(w, 0xFF),
         jnp.bitwise_and(jnp.right_shift(w, 16), 0xFF)], axis=0
    ).astype(jnp.float32)
    c1 = jnp.concatenate(
        [jnp.bitwise_and(jnp.right_shift(w, 8), 0xFF),
         jnp.right_shift(w, 24)], axis=0
    ).astype(jnp.float32)
    dn = (((1,), (0,)), ((), ()))
    sum0 = jax.lax.dot_general(c0, emb0_ref[...], dn,
                               preferred_element_type=jnp.float32)
    sum1 = jax.lax.dot_general(c1, emb1_ref[...], dn,
                               preferred_element_type=jnp.float32)
    h0 = jnp.maximum(sum0, 0.0)
    h1 = jnp.maximum(sum1, 0.0)
    dn_nt = (((1,), (1,)), ((), ()))
    w2 = w2_ref[...]
    x = (jax.lax.dot_general(h0, w2[:, :HIDDEN], dn_nt,
                             preferred_element_type=jnp.float32)
         + jax.lax.dot_general(h1, w2[:, HIDDEN:], dn_nt,
                               preferred_element_type=jnp.float32)
         + b2_ref[...])
    x = jnp.maximum(x, 0.0)
    x = jax.lax.dot_general(x, w3_ref[...], dn_nt,
                            preferred_element_type=jnp.float32) + b3_ref[...]
    x = jnp.maximum(x, 0.0)
    out_ref[...] = (jax.lax.dot_general(w4_ref[...], x, dn_nt,
                                        preferred_element_type=jnp.float32)
                    + b4_ref[0, 0])


@jax.jit
def kernel(idx0_batch, idx1_batch, emb0_w, emb1_w, fc2_w, fc2_b, fc3_w,
           fc3_b, fc4_w, fc4_b):
    counts = _sc_hist(idx0_batch.astype(jnp.int32),
                      idx1_batch.astype(jnp.int32))

    b2 = fc2_b.reshape(1, -1)
    b3 = fc3_b.reshape(1, -1)
    b4 = fc4_b.reshape(1, 1)
    ntile = (B // 2) // BT2
    out = pl.pallas_call(
        _tc_kernel,
        grid=(ntile,),
        in_specs=[
            pl.BlockSpec((BT2, TABLE_SIZE), lambda i: (i, 0)),
            pl.BlockSpec((TABLE_SIZE, HIDDEN), lambda i: (0, 0)),
            pl.BlockSpec((TABLE_SIZE, HIDDEN), lambda i: (0, 0)),
            pl.BlockSpec(fc2_w.shape, lambda i: (0, 0)),
            pl.BlockSpec(b2.shape, lambda i: (0, 0)),
            pl.BlockSpec(fc3_w.shape, lambda i: (0, 0)),
            pl.BlockSpec(b3.shape, lambda i: (0, 0)),
            pl.BlockSpec(fc4_w.shape, lambda i: (0, 0)),
            pl.BlockSpec(b4.shape, lambda i: (0, 0)),
        ],
        out_specs=pl.BlockSpec((1, 2 * BT2), lambda i: (0, i)),
        out_shape=jax.ShapeDtypeStruct((1, ntile * 2 * BT2), jnp.float32),
        compiler_params=pltpu.CompilerParams(
            dimension_semantics=("arbitrary",),
        ),
    )(counts, emb0_w, emb1_w, fc2_w, b2, fc3_w, b3, fc4_w, b4)
    return out.reshape(ntile, 2, BT2 // PR, PR).transpose(0, 2, 3, 1).reshape(B)


# speedup vs baseline: 1.2105x; 1.2105x over previous
import functools

import jax
import jax.numpy as jnp
from jax import lax
from jax.experimental import pallas as pl
from jax.experimental.pallas import tpu as pltpu
from jax.experimental.pallas import tpu_sc as plsc

TABLE_SIZE = 640
HIDDEN = 256
B = 4096
L = 50

_info = plsc.get_sparse_core_info()
NC, NS = _info.num_cores, _info.num_subcores
NW = NC * NS
SPT = B // NW
QS = 64
NQ = SPT // QS
PR = QS // 2

_mesh = plsc.VectorSubcoreMesh(core_axis_name="c", subcore_axis_name="s")


@functools.partial(
    pl.kernel,
    mesh=_mesh,
    out_type=jax.ShapeDtypeStruct((B // 2, TABLE_SIZE), jnp.int32),
    scratch_types=[
        pltpu.VMEM((L, SPT), jnp.int32),
        pltpu.VMEM((L, SPT), jnp.int32),
        pltpu.VMEM((PR, TABLE_SIZE), jnp.int32),
        pltpu.VMEM((PR, TABLE_SIZE), jnp.int32),
        pltpu.SemaphoreType.DMA,
        pltpu.SemaphoreType.DMA,
        pltpu.SemaphoreType.DMA,
    ],
    compiler_params=pltpu.CompilerParams(needs_layout_passes=False),
)
def _sc_hist(idx0_hbm, idx1_hbm, out_hbm, idx0_v, idx1_v, cnt_a, cnt_b,
             sem_i, sem_a, sem_b):
    wid = lax.axis_index("s") * NC + lax.axis_index("c")
    sbase = wid * SPT
    pbase = wid * (SPT // 2)
    zero16 = jnp.zeros((16,), jnp.int32)
    adds = [[jnp.full((16,), 1 << (16 * f + 8 * t), jnp.int32)
             for t in range(2)] for f in range(2)]
    lane = lax.iota(jnp.int32, 16)
    ld0 = pltpu.async_copy(idx0_hbm.at[:, pl.ds(sbase, SPT)], idx0_v, sem_i)
    ld1 = pltpu.async_copy(idx1_hbm.at[:, pl.ds(sbase, SPT)], idx1_v, sem_i)
    ld0.wait()
    ld1.wait()

    copies = []
    for q in range(NQ):
        cnt_v = (cnt_a, cnt_b)[q % 2]
        sem_o = (sem_a, sem_b)[q % 2]

        def zrow_body(s, zc, cnt_v=cnt_v):
            for k in range(TABLE_SIZE // 16):
                cnt_v[s, pl.ds(k * 16, 16)] = zero16
            return zc

        lax.fori_loop(0, PR, zrow_body, 0)

        for m in range(4):
            f = m % 2
            col = q * QS + m * 16
            row16 = lane + (m // 2) * 16
            a0, a1 = adds[f][0], adds[f][1]

            def j_body(j5, jc, col=col, row16=row16, a0=a0, a1=a1,
                       cnt_v=cnt_v):
                for u in range(5):
                    j = j5 * 5 + u
                    cv0 = idx0_v[j, pl.ds(col, 16)]
                    plsc.addupdate_scatter(cnt_v, [row16, cv0], a0)
                    cv1 = idx1_v[j, pl.ds(col, 16)]
                    plsc.addupdate_scatter(cnt_v, [row16, cv1], a1)
                return jc

            lax.fori_loop(0, L // 5, j_body, 0)

        copies.append(pltpu.async_copy(
            cnt_v, out_hbm.at[pl.ds(pbase + q * PR, PR)], sem_o))
    for cp in copies:
        cp.wait()


BT2 = 512


def _tc_kernel(cnt_ref, emb0_ref, emb1_ref, w2_ref, b2_ref,
               w3_ref, b3_ref, w4_ref, b4_ref, out_ref):
    w = cnt_ref[...]
    c0 = jnp.concatenate(
        [jnp.bitwise_and(w, 0xFF),
         jnp.bitwise_and(jnp.right_shift(w, 16), 0xFF)], axis=0
    ).astype(jnp.float32)
    c1 = jnp.concatenate(
        [jnp.bitwise_and(jnp.right_shift(w, 8), 0xFF),
         jnp.right_shift(w, 24)], axis=0
    ).astype(jnp.float32)
    dn = (((1,), (0,)), ((), ()))
    sum0 = jax.lax.dot_general(c0, emb0_ref[...], dn,
                               preferred_element_type=jnp.float32)
    sum1 = jax.lax.dot_general(c1, emb1_ref[...], dn,
                               preferred_element_type=jnp.float32)
    h0 = jnp.maximum(sum0, 0.0)
    h1 = jnp.maximum(sum1, 0.0)
    dn_nt = (((1,), (1,)), ((), ()))
    w2 = w2_ref[...]
    x = (jax.lax.dot_general(h0, w2[:, :HIDDEN], dn_nt,
                             preferred_element_type=jnp.float32)
         + jax.lax.dot_general(h1, w2[:, HIDDEN:], dn_nt,
                               preferred_element_type=jnp.float32)
         + b2_ref[...])
    x = jnp.maximum(x, 0.0)
    x = jax.lax.dot_general(x, w3_ref[...], dn_nt,
                            preferred_element_type=jnp.float32) + b3_ref[...]
    x = jnp.maximum(x, 0.0)
    out_ref[...] = (jax.lax.dot_general(w4_ref[...], x, dn_nt,
                                        preferred_element_type=jnp.float32)
                    + b4_ref[0, 0])


@jax.jit
def kernel(idx0_batch, idx1_batch, emb0_w, emb1_w, fc2_w, fc2_b, fc3_w,
           fc3_b, fc4_w, fc4_b):
    idx0_t = idx0_batch.astype(jnp.int32).T
    idx1_t = idx1_batch.astype(jnp.int32).T
    counts = _sc_hist(idx0_t, idx1_t)

    b2 = fc2_b.reshape(1, -1)
    b3 = fc3_b.reshape(1, -1)
    b4 = fc4_b.reshape(1, 1)
    ntile = (B // 2) // BT2
    out = pl.pallas_call(
        _tc_kernel,
        grid=(ntile,),
        in_specs=[
            pl.BlockSpec((BT2, TABLE_SIZE), lambda i: (i, 0)),
            pl.BlockSpec((TABLE_SIZE, HIDDEN), lambda i: (0, 0)),
            pl.BlockSpec((TABLE_SIZE, HIDDEN), lambda i: (0, 0)),
            pl.BlockSpec(fc2_w.shape, lambda i: (0, 0)),
            pl.BlockSpec(b2.shape, lambda i: (0, 0)),
            pl.BlockSpec(fc3_w.shape, lambda i: (0, 0)),
            pl.BlockSpec(b3.shape, lambda i: (0, 0)),
            pl.BlockSpec(fc4_w.shape, lambda i: (0, 0)),
            pl.BlockSpec(b4.shape, lambda i: (0, 0)),
        ],
        out_specs=pl.BlockSpec((1, 2 * BT2), lambda i: (0, i)),
        out_shape=jax.ShapeDtypeStruct((1, ntile * 2 * BT2), jnp.float32),
        compiler_params=pltpu.CompilerParams(
            dimension_semantics=("arbitrary",),
        ),
    )(counts, emb0_w, emb1_w, fc2_w, b2, fc3_w, b3, fc4_w, b4)
    return (out.reshape(ntile, 2, 16, 2, 16)
            .transpose(0, 2, 3, 1, 4).reshape(B))


# speedup vs baseline: 1.2539x; 1.0359x over previous
import functools

import jax
import jax.numpy as jnp
from jax import lax
from jax.experimental import pallas as pl
from jax.experimental.pallas import tpu as pltpu
from jax.experimental.pallas import tpu_sc as plsc

TABLE_SIZE = 640
HIDDEN = 256
B = 4096
L = 50

_info = plsc.get_sparse_core_info()
NC, NS = _info.num_cores, _info.num_subcores
NW = NC * NS
SPT = B // NW
QS = 64
NQ = SPT // QS
PR = QS // 2

_mesh = plsc.VectorSubcoreMesh(core_axis_name="c", subcore_axis_name="s")


@functools.partial(
    pl.kernel,
    mesh=_mesh,
    out_type=jax.ShapeDtypeStruct((B // 2, TABLE_SIZE), jnp.int32),
    scratch_types=[
        pltpu.VMEM((L, SPT), jnp.int32),
        pltpu.VMEM((L, SPT), jnp.int32),
        pltpu.VMEM((PR, TABLE_SIZE), jnp.int32),
        pltpu.VMEM((PR, TABLE_SIZE), jnp.int32),
        pltpu.SemaphoreType.DMA,
        pltpu.SemaphoreType.DMA,
        pltpu.SemaphoreType.DMA,
    ],
    compiler_params=pltpu.CompilerParams(needs_layout_passes=False),
)
def _sc_hist(idx0_hbm, idx1_hbm, out_hbm, idx0_v, idx1_v, cnt_a, cnt_b,
             sem_i, sem_a, sem_b):
    wid = lax.axis_index("s") * NC + lax.axis_index("c")
    sbase = wid * SPT
    pbase = wid * (SPT // 2)
    zero16 = jnp.zeros((16,), jnp.int32)
    adds = [[jnp.full((16,), 1 << (16 * f + 8 * t), jnp.int32)
             for t in range(2)] for f in range(2)]
    lane = lax.iota(jnp.int32, 16)
    ld0 = pltpu.async_copy(idx0_hbm.at[:, pl.ds(sbase, SPT)], idx0_v, sem_i)
    ld1 = pltpu.async_copy(idx1_hbm.at[:, pl.ds(sbase, SPT)], idx1_v, sem_i)

    for buf in (cnt_a, cnt_b):

        def zrow_body(s, zc, buf=buf):
            for k in range(TABLE_SIZE // 16):
                buf[s, pl.ds(k * 16, 16)] = zero16
            return zc

        lax.fori_loop(0, PR, zrow_body, 0)

    ld0.wait()
    ld1.wait()

    copies = []
    for q in range(NQ):
        cnt_v = (cnt_a, cnt_b)[q % 2]
        sem_o = (sem_a, sem_b)[q % 2]

        for m in range(4):
            f = m % 2
            col = q * QS + m * 16
            row16 = lane + (m // 2) * 16
            a0, a1 = adds[f][0], adds[f][1]

            def j_body(j5, jc, col=col, row16=row16, a0=a0, a1=a1,
                       cnt_v=cnt_v):
                for u in range(5):
                    j = j5 * 5 + u
                    cv0 = idx0_v[j, pl.ds(col, 16)]
                    plsc.addupdate_scatter(cnt_v, [row16, cv0], a0)
                    cv1 = idx1_v[j, pl.ds(col, 16)]
                    plsc.addupdate_scatter(cnt_v, [row16, cv1], a1)
                return jc

            lax.fori_loop(0, L // 5, j_body, 0)

        copies.append(pltpu.async_copy(
            cnt_v, out_hbm.at[pl.ds(pbase + q * PR, PR)], sem_o))
    for cp in copies:
        cp.wait()


BT2 = 512


def _tc_kernel(cnt_ref, emb0_ref, emb1_ref, w2_ref, b2_ref,
               w3_ref, b3_ref, w4_ref, b4_ref, out_ref):
    w = cnt_ref[...]
    c0 = jnp.concatenate(
        [jnp.bitwise_and(w, 0xFF),
         jnp.bitwise_and(jnp.right_shift(w, 16), 0xFF)], axis=0
    ).astype(jnp.float32)
    c1 = jnp.concatenate(
        [jnp.bitwise_and(jnp.right_shift(w, 8), 0xFF),
         jnp.right_shift(w, 24)], axis=0
    ).astype(jnp.float32)
    dn = (((1,), (0,)), ((), ()))
    sum0 = jax.lax.dot_general(c0, emb0_ref[...], dn,
                               preferred_element_type=jnp.float32)
    sum1 = jax.lax.dot_general(c1, emb1_ref[...], dn,
                               preferred_element_type=jnp.float32)
    h0 = jnp.maximum(sum0, 0.0)
    h1 = jnp.maximum(sum1, 0.0)
    dn_nt = (((1,), (1,)), ((), ()))
    w2 = w2_ref[...]
    x = (jax.lax.dot_general(h0, w2[:, :HIDDEN], dn_nt,
                             preferred_element_type=jnp.float32)
         + jax.lax.dot_general(h1, w2[:, HIDDEN:], dn_nt,
                               preferred_element_type=jnp.float32)
         + b2_ref[...])
    x = jnp.maximum(x, 0.0)
    x = jax.lax.dot_general(x, w3_ref[...], dn_nt,
                            preferred_element_type=jnp.float32) + b3_ref[...]
    x = jnp.maximum(x, 0.0)
    out_ref[...] = (jax.lax.dot_general(w4_ref[...], x, dn_nt,
                                        preferred_element_type=jnp.float32)
                    + b4_ref[0, 0])


@jax.jit
def kernel(idx0_batch, idx1_batch, emb0_w, emb1_w, fc2_w, fc2_b, fc3_w,
           fc3_b, fc4_w, fc4_b):
    idx0_t = idx0_batch.astype(jnp.int32).T
    idx1_t = idx1_batch.astype(jnp.int32).T
    counts = _sc_hist(idx0_t, idx1_t)

    b2 = fc2_b.reshape(1, -1)
    b3 = fc3_b.reshape(1, -1)
    b4 = fc4_b.reshape(1, 1)
    ntile = (B // 2) // BT2
    out = pl.pallas_call(
        _tc_kernel,
        grid=(ntile,),
        in_specs=[
            pl.BlockSpec((BT2, TABLE_SIZE), lambda i: (i, 0)),
            pl.BlockSpec((TABLE_SIZE, HIDDEN), lambda i: (0, 0)),
            pl.BlockSpec((TABLE_SIZE, HIDDEN), lambda i: (0, 0)),
            pl.BlockSpec(fc2_w.shape, lambda i: (0, 0)),
            pl.BlockSpec(b2.shape, lambda i: (0, 0)),
            pl.BlockSpec(fc3_w.shape, lambda i: (0, 0)),
            pl.BlockSpec(b3.shape, lambda i: (0, 0)),
            pl.BlockSpec(fc4_w.shape, lambda i: (0, 0)),
            pl.BlockSpec(b4.shape, lambda i: (0, 0)),
        ],
        out_specs=pl.BlockSpec((1, 2 * BT2), lambda i: (0, i)),
        out_shape=jax.ShapeDtypeStruct((1, ntile * 2 * BT2), jnp.float32),
        compiler_params=pltpu.CompilerParams(
            dimension_semantics=("arbitrary",),
        ),
    )(counts, emb0_w, emb1_w, fc2_w, b2, fc3_w, b3, fc4_w, b4)
    return (out.reshape(ntile, 2, 16, 2, 16)
            .transpose(0, 2, 3, 1, 4).reshape(B))


# speedup vs baseline: 1.2704x; 1.0132x over previous
import functools

import jax
import jax.numpy as jnp
from jax import lax
from jax.experimental import pallas as pl
from jax.experimental.pallas import tpu as pltpu
from jax.experimental.pallas import tpu_sc as plsc

TABLE_SIZE = 640
HIDDEN = 256
B = 4096
L = 50

_info = plsc.get_sparse_core_info()
NC, NS = _info.num_cores, _info.num_subcores
NW = NC * NS
SPT = B // NW
QS = 64
NQ = SPT // QS
PR = QS // 2

_mesh = plsc.VectorSubcoreMesh(core_axis_name="c", subcore_axis_name="s")


@functools.partial(
    pl.kernel,
    mesh=_mesh,
    out_type=jax.ShapeDtypeStruct((B // 2, TABLE_SIZE), jnp.int32),
    scratch_types=[
        pltpu.VMEM((L, SPT), jnp.int32),
        pltpu.VMEM((L, SPT), jnp.int32),
        pltpu.VMEM((PR, TABLE_SIZE), jnp.int32),
        pltpu.VMEM((PR, TABLE_SIZE), jnp.int32),
        pltpu.SemaphoreType.DMA,
        pltpu.SemaphoreType.DMA,
        pltpu.SemaphoreType.DMA,
    ],
    compiler_params=pltpu.CompilerParams(needs_layout_passes=False),
)
def _sc_hist(idx0_hbm, idx1_hbm, out_hbm, idx0_v, idx1_v, cnt_a, cnt_b,
             sem_i, sem_a, sem_b):
    wid = lax.axis_index("s") * NC + lax.axis_index("c")
    sbase = wid * SPT
    pbase = wid * (SPT // 2)
    zero16 = jnp.zeros((16,), jnp.int32)
    adds = [[jnp.full((16,), 1 << (16 * f + 8 * t), jnp.int32)
             for t in range(2)] for f in range(2)]
    lane = lax.iota(jnp.int32, 16)
    ld0 = pltpu.async_copy(idx0_hbm.at[:, pl.ds(sbase, SPT)], idx0_v, sem_i)
    ld1 = pltpu.async_copy(idx1_hbm.at[:, pl.ds(sbase, SPT)], idx1_v, sem_i)

    for buf in (cnt_a, cnt_b):

        def zrow_body(s, zc, buf=buf):
            for k in range(TABLE_SIZE // 16):
                buf[s, pl.ds(k * 16, 16)] = zero16
            return zc

        lax.fori_loop(0, PR, zrow_body, 0)

    ld0.wait()
    ld1.wait()

    copies = []
    for q in range(NQ):
        cnt_v = (cnt_a, cnt_b)[q % 2]
        sem_o = (sem_a, sem_b)[q % 2]

        for m in range(4):
            f = m % 2
            col = q * QS + m * 16
            row16 = lane + (m // 2) * 16
            a0, a1 = adds[f][0], adds[f][1]

            def j_body(j5, jc, col=col, row16=row16, a0=a0, a1=a1,
                       cnt_v=cnt_v):
                for u in range(5):
                    j = j5 * 5 + u
                    cv0 = idx0_v[j, pl.ds(col, 16)]
                    plsc.addupdate_scatter(cnt_v, [row16, cv0], a0)
                    cv1 = idx1_v[j, pl.ds(col, 16)]
                    plsc.addupdate_scatter(cnt_v, [row16, cv1], a1)
                return jc

            lax.fori_loop(0, L // 5, j_body, 0)

        copies.append(pltpu.async_copy(
            cnt_v, out_hbm.at[pl.ds(pbase + q * PR, PR)], sem_o))
    for cp in copies:
        cp.wait()


BT2 = 1024


def _tc_kernel(cnt_ref, emb0_ref, emb1_ref, w2_ref, b2_ref,
               w3_ref, b3_ref, w4_ref, b4_ref, out_ref):
    w = cnt_ref[...]
    c0 = jnp.concatenate(
        [jnp.bitwise_and(w, 0xFF),
         jnp.bitwise_and(jnp.right_shift(w, 16), 0xFF)], axis=0
    ).astype(jnp.float32)
    c1 = jnp.concatenate(
        [jnp.bitwise_and(jnp.right_shift(w, 8), 0xFF),
         jnp.right_shift(w, 24)], axis=0
    ).astype(jnp.float32)
    dn = (((1,), (0,)), ((), ()))
    sum0 = jax.lax.dot_general(c0, emb0_ref[...], dn,
                               preferred_element_type=jnp.float32)
    sum1 = jax.lax.dot_general(c1, emb1_ref[...], dn,
                               preferred_element_type=jnp.float32)
    h0 = jnp.maximum(sum0, 0.0)
    h1 = jnp.maximum(sum1, 0.0)
    dn_nt = (((1,), (1,)), ((), ()))
    w2 = w2_ref[...]
    x = (jax.lax.dot_general(h0, w2[:, :HIDDEN], dn_nt,
                             preferred_element_type=jnp.float32)
         + jax.lax.dot_general(h1, w2[:, HIDDEN:], dn_nt,
                               preferred_element_type=jnp.float32)
         + b2_ref[...])
    x = jnp.maximum(x, 0.0)
    x = jax.lax.dot_general(x, w3_ref[...], dn_nt,
                            preferred_element_type=jnp.float32) + b3_ref[...]
    x = jnp.maximum(x, 0.0)
    out_ref[...] = (jax.lax.dot_general(w4_ref[...], x, dn_nt,
                                        preferred_element_type=jnp.float32)
                    + b4_ref[0, 0])


@jax.jit
def kernel(idx0_batch, idx1_batch, emb0_w, emb1_w, fc2_w, fc2_b, fc3_w,
           fc3_b, fc4_w, fc4_b):
    idx0_t = idx0_batch.astype(jnp.int32).T
    idx1_t = idx1_batch.astype(jnp.int32).T
    counts = _sc_hist(idx0_t, idx1_t)

    b2 = fc2_b.reshape(1, -1)
    b3 = fc3_b.reshape(1, -1)
    b4 = fc4_b.reshape(1, 1)
    ntile = (B // 2) // BT2
    out = pl.pallas_call(
        _tc_kernel,
        grid=(ntile,),
        in_specs=[
            pl.BlockSpec((BT2, TABLE_SIZE), lambda i: (i, 0)),
            pl.BlockSpec((TABLE_SIZE, HIDDEN), lambda i: (0, 0)),
            pl.BlockSpec((TABLE_SIZE, HIDDEN), lambda i: (0, 0)),
            pl.BlockSpec(fc2_w.shape, lambda i: (0, 0)),
            pl.BlockSpec(b2.shape, lambda i: (0, 0)),
            pl.BlockSpec(fc3_w.shape, lambda i: (0, 0)),
            pl.BlockSpec(b3.shape, lambda i: (0, 0)),
            pl.BlockSpec(fc4_w.shape, lambda i: (0, 0)),
            pl.BlockSpec(b4.shape, lambda i: (0, 0)),
        ],
        out_specs=pl.BlockSpec((1, 2 * BT2), lambda i: (0, i)),
        out_shape=jax.ShapeDtypeStruct((1, ntile * 2 * BT2), jnp.float32),
        compiler_params=pltpu.CompilerParams(
            dimension_semantics=("arbitrary",),
        ),
    )(counts, emb0_w, emb1_w, fc2_w, b2, fc3_w, b3, fc4_w, b4)
    return (out.reshape(ntile, 2, BT2 // PR, 2, 16)
            .transpose(0, 2, 3, 1, 4).reshape(B))
